# baseline (speedup 1.0000x reference)
import functools

import jax
import jax.numpy as jnp
from jax import lax
from jax.experimental import pallas as pl
from jax.experimental.pallas import tpu as pltpu
from jax.experimental.pallas import tpu_sc as plsc

_B, _N, _D = 128, 8192, 16
_NW = 32
_RPW = _B // _NW
_CHUNK = 2048
_NCH = _N // _CHUNK
_CHF = _CHUNK * _D
_TILES = _CHUNK // 16
_NG = _RPW * _NCH


def _body(x_hbm, o_hbm, buf0, buf1, win, sem0, sem1):
    wid = lax.axis_index("s") * 2 + lax.axis_index("c")
    iota = lax.iota(jnp.int32, 16)
    cols = [iota * _D + k for k in range(_D)]

    bufs = (buf0, buf1)
    sems = (sem0, sem1)

    def start(g):
        row = wid * _RPW + (g // _NCH)
        off = row * (_N * _D) + (g % _NCH) * _CHF
        return pltpu.async_copy(
            x_hbm.at[pl.ds(off, _CHF)], bufs[g % 2], sems[g % 2])

    cur = start(0)
    bv = bi = None
    for g in range(_NG):
        nxt = start(g + 1) if g + 1 < _NG else None
        cur.wait()
        buf = bufs[g % 2]
        c = g % _NCH
        if c == 0:
            bv = jnp.full((16,), -1.0, jnp.float32)
            bi = jnp.zeros((16,), jnp.int32)
        cbase = c * _CHUNK

        def tile(t, carry, buf=buf, cbase=cbase):
            tbv, tbi = carry
            base = t * (16 * _D)
            acc = jnp.zeros((16,), jnp.float32)
            for k in range(_D):
                v = plsc.load_gather(buf, [cols[k] + base])
                acc = acc + v * v
            idxv = iota + (cbase + t * 16)
            m = acc > tbv
            return jnp.where(m, acc, tbv), jnp.where(m, idxv, tbi)

        bv, bi = lax.fori_loop(0, _TILES, tile, (bv, bi))

        if c == _NCH - 1:
            row = wid * _RPW + (g // _NCH)
            mx = jnp.max(bv)
            cand = jnp.where(bv == mx, bi, jnp.int32(_N))
            j = jnp.min(cand)
            off = row * (_N * _D) + j * _D
            pltpu.sync_copy(x_hbm.at[pl.ds(off, _D)], win)
            pltpu.sync_copy(win, o_hbm.at[pl.ds(row * _D, _D)])
        cur = nxt


_mask_cid = functools.partial(
    pl.kernel,
    out_type=jax.ShapeDtypeStruct((_B * _D,), jnp.float32),
    mesh=plsc.VectorSubcoreMesh(core_axis_name="c", subcore_axis_name="s"),
    compiler_params=pltpu.CompilerParams(needs_layout_passes=False),
    scratch_types=[
        pltpu.VMEM((_CHF,), jnp.float32),
        pltpu.VMEM((_CHF,), jnp.float32),
        pltpu.VMEM((_D,), jnp.float32),
        pltpu.SemaphoreType.DMA,
        pltpu.SemaphoreType.DMA,
    ],
)(_body)


@jax.jit
def kernel(inputs):
    out = _mask_cid(inputs.reshape(-1))
    return out.reshape(_B, _D)


# speedup vs baseline: 1.0469x; 1.0469x over previous
import functools

import jax
import jax.numpy as jnp
from jax import lax
from jax.experimental import pallas as pl
from jax.experimental.pallas import tpu as pltpu
from jax.experimental.pallas import tpu_sc as plsc

_B, _N, _D = 128, 8192, 16
_NW = 32
_RPW = _B // _NW
_CHUNK = 2048
_NCH = _N // _CHUNK
_CHF = _CHUNK * _D
_TILES = _CHUNK // 16
_NG = _RPW * _NCH
_UNR = 4


def _body(x_hbm, o_hbm, buf0, buf1, win, sem0, sem1):
    wid = lax.axis_index("s") * 2 + lax.axis_index("c")
    iota = lax.iota(jnp.int32, 16)
    diag = [iota * _D + ((iota + k) & (_D - 1)) for k in range(_D)]

    bufs = (buf0, buf1)
    sems = (sem0, sem1)

    def start(g):
        row = wid * _RPW + (g // _NCH)
        off = row * (_N * _D) + (g % _NCH) * _CHF
        return pltpu.async_copy(
            x_hbm.at[pl.ds(off, _CHF)], bufs[g % 2], sems[g % 2])

    cur = start(0)
    bv = bi = None
    for g in range(_NG):
        nxt = start(g + 1) if g + 1 < _NG else None
        cur.wait()
        buf = bufs[g % 2]
        c = g % _NCH
        if c == 0:
            bv = [jnp.full((16,), -1.0, jnp.float32) for _ in range(_UNR)]
            bi = [jnp.zeros((16,), jnp.int32) for _ in range(_UNR)]
        cbase = c * _CHUNK

        def group(q, carry, buf=buf, cbase=cbase):
            sbv = list(carry[: _UNR])
            sbi = list(carry[_UNR:])
            for u in range(_UNR):
                base = (q * _UNR + u) * (16 * _D)
                vs = [plsc.load_gather(buf, [diag[k] + base])
                      for k in range(_D)]
                sq = [v * v for v in vs]
                while len(sq) > 1:
                    sq = [sq[i] + sq[i + 1] for i in range(0, len(sq), 2)]
                acc = sq[0]
                idxv = iota + (cbase + (q * _UNR + u) * 16)
                m = acc > sbv[u]
                sbv[u] = jnp.where(m, acc, sbv[u])
                sbi[u] = jnp.where(m, idxv, sbi[u])
            return tuple(sbv) + tuple(sbi)

        res = lax.fori_loop(0, _TILES // _UNR, group, tuple(bv) + tuple(bi))
        bv = list(res[: _UNR])
        bi = list(res[_UNR:])

        if c == _NCH - 1:
            row = wid * _RPW + (g // _NCH)
            mv, mi = bv[0], bi[0]
            for u in range(1, _UNR):
                take = (bv[u] > mv) | ((bv[u] == mv) & (bi[u] < mi))
                mv = jnp.where(take, bv[u], mv)
                mi = jnp.where(take, bi[u], mi)
            mx = jnp.max(mv)
            cand = jnp.where(mv == mx, mi, jnp.int32(_N))
            j = jnp.min(cand)
            off = row * (_N * _D) + j * _D
            pltpu.sync_copy(x_hbm.at[pl.ds(off, _D)], win)
            pltpu.sync_copy(win, o_hbm.at[pl.ds(row * _D, _D)])
        cur = nxt


_mask_cid = functools.partial(
    pl.kernel,
    out_type=jax.ShapeDtypeStruct((_B * _D,), jnp.float32),
    mesh=plsc.VectorSubcoreMesh(core_axis_name="c", subcore_axis_name="s"),
    compiler_params=pltpu.CompilerParams(needs_layout_passes=False),
    scratch_types=[
        pltpu.VMEM((_CHF,), jnp.float32),
        pltpu.VMEM((_CHF,), jnp.float32),
        pltpu.VMEM((_D,), jnp.float32),
        pltpu.SemaphoreType.DMA,
        pltpu.SemaphoreType.DMA,
    ],
)(_body)


@jax.jit
def kernel(inputs):
    out = _mask_cid(inputs.reshape(-1))
    return out.reshape(_B, _D)


# speedup vs baseline: 1.0539x; 1.0068x over previous
import functools

import jax
import jax.numpy as jnp
from jax import lax
from jax.experimental import pallas as pl
from jax.experimental.pallas import tpu as pltpu
from jax.experimental.pallas import tpu_sc as plsc

_B, _N, _D = 128, 8192, 16
_NW = 32
_RPW = _B // _NW
_CHUNK = 2048
_NCH = _N // _CHUNK
_TILES = _CHUNK // 16
_NG = _RPW * _NCH
_UNR = 4


def _body(x_hbm, o_hbm, buf0, buf1, win, sem0, sem1):
    wid = lax.axis_index("s") * 2 + lax.axis_index("c")
    iota = lax.iota(jnp.int32, 16)
    diag = [(iota + k) & (_D - 1) for k in range(_D)]

    bufs = (buf0, buf1)
    sems = (sem0, sem1)

    def start(g):
        row = wid * _RPW + (g // _NCH)
        cap0 = (g % _NCH) * _CHUNK
        return pltpu.async_copy(
            x_hbm.at[row, pl.ds(cap0, _CHUNK)], bufs[g % 2], sems[g % 2])

    cur = start(0)
    bv = bi = None
    for g in range(_NG):
        nxt = start(g + 1) if g + 1 < _NG else None
        cur.wait()
        buf = bufs[g % 2]
        c = g % _NCH
        if c == 0:
            bv = [jnp.full((16,), -1.0, jnp.float32) for _ in range(_UNR)]
            bi = [jnp.zeros((16,), jnp.int32) for _ in range(_UNR)]
        cbase = c * _CHUNK

        def group(q, carry, buf=buf, cbase=cbase):
            sbv = list(carry[: _UNR])
            sbi = list(carry[_UNR:])
            for u in range(_UNR):
                cap = iota + (q * _UNR + u) * 16
                vs = [plsc.load_gather(buf, [cap, diag[k]])
                      for k in range(_D)]
                sq = [v * v for v in vs]
                while len(sq) > 1:
                    sq = [sq[i] + sq[i + 1] for i in range(0, len(sq), 2)]
                acc = sq[0]
                idxv = cap + cbase
                m = acc > sbv[u]
                sbv[u] = jnp.where(m, acc, sbv[u])
                sbi[u] = jnp.where(m, idxv, sbi[u])
            return tuple(sbv) + tuple(sbi)

        res = lax.fori_loop(0, _TILES // _UNR, group, tuple(bv) + tuple(bi))
        bv = list(res[: _UNR])
        bi = list(res[_UNR:])

        if c == _NCH - 1:
            row = wid * _RPW + (g // _NCH)
            mv, mi = bv[0], bi[0]
            for u in range(1, _UNR):
                take = (bv[u] > mv) | ((bv[u] == mv) & (bi[u] < mi))
                mv = jnp.where(take, bv[u], mv)
                mi = jnp.where(take, bi[u], mi)
            mx = jnp.max(mv)
            cand = jnp.where(mv == mx, mi, jnp.int32(_N))
            j = jnp.min(cand)
            pltpu.sync_copy(x_hbm.at[row, pl.ds(j, 1)], win)
            pltpu.sync_copy(win, o_hbm.at[pl.ds(row, 1)])
        cur = nxt


_mask_cid = functools.partial(
    pl.kernel,
    out_type=jax.ShapeDtypeStruct((_B, _D), jnp.float32),
    mesh=plsc.VectorSubcoreMesh(core_axis_name="c", subcore_axis_name="s"),
    compiler_params=pltpu.CompilerParams(
        needs_layout_passes=False, use_tc_tiling_on_sc=False),
    scratch_types=[
        pltpu.VMEM((_CHUNK, _D), jnp.float32),
        pltpu.VMEM((_CHUNK, _D), jnp.float32),
        pltpu.VMEM((1, _D), jnp.float32),
        pltpu.SemaphoreType.DMA,
        pltpu.SemaphoreType.DMA,
    ],
)(_body)


@jax.jit
def kernel(inputs):
    return _mask_cid(inputs)


# speedup vs baseline: 8.3910x; 7.9617x over previous
import functools

import jax
import jax.numpy as jnp
from jax import lax
from jax.experimental import pallas as pl
from jax.experimental.pallas import tpu as pltpu
from jax.experimental.pallas import tpu_sc as plsc

_B, _N, _D = 128, 8192, 16
_NW = 32
_RPW = _B // _NW
_CW = 2048
_NCH = _N // _CW
_COLS = _CW // 128
_NG = _RPW * _NCH
_UNR = 2


def _body(x_hbm, o_hbm, buf0, buf1, win, wout, sem0, sem1):
    wid = lax.axis_index("s") * 2 + lax.axis_index("c")
    iota = lax.iota(jnp.int32, 16)

    bufs = (buf0, buf1)
    sems = (sem0, sem1)

    def start(g):
        row = wid * _RPW + (g // _NCH)
        cap0 = (g % _NCH) * _CW
        return pltpu.async_copy(
            x_hbm.at[row, :, pl.ds(cap0, _CW)], bufs[g % 2], sems[g % 2])

    cur = start(0)
    bv = bi = None
    for g in range(_NG):
        nxt = start(g + 1) if g + 1 < _NG else None
        cur.wait()
        buf = bufs[g % 2]
        c = g % _NCH
        if c == 0:
            bv = [jnp.full((16,), -1.0, jnp.float32) for _ in range(_UNR)]
            bi = [jnp.zeros((16,), jnp.int32) for _ in range(_UNR)]
        cbase = c * _CW

        def grp(q, carry, buf=buf, cbase=cbase):
            sbv = list(carry[: _UNR])
            sbi = list(carry[_UNR:])
            for m in range(_UNR):
                vs = [buf[k, pl.ds(q * (16 * _UNR) + m * 16, 16)]
                      for k in range(_D)]
                sq = [v * v for v in vs]
                while len(sq) > 1:
                    sq = [sq[i] + sq[i + 1] for i in range(0, len(sq), 2)]
                acc = sq[0]
                idxv = iota + (cbase + m * 16) + q * (16 * _UNR)
                m_upd = acc > sbv[m]
                sbv[m] = jnp.where(m_upd, acc, sbv[m])
                sbi[m] = jnp.where(m_upd, idxv, sbi[m])
            return tuple(sbv) + tuple(sbi)

        res = lax.fori_loop(0, _CW // (16 * _UNR), grp,
                            tuple(bv) + tuple(bi))
        bv = list(res[: _UNR])
        bi = list(res[_UNR:])

        if c == _NCH - 1:
            row = wid * _RPW + (g // _NCH)
            mv, mi = bv[0], bi[0]
            for u in range(1, _UNR):
                take = (bv[u] > mv) | ((bv[u] == mv) & (bi[u] < mi))
                mv = jnp.where(take, bv[u], mv)
                mi = jnp.where(take, bi[u], mi)
            mx = jnp.max(mv)
            cand = jnp.where(mv == mx, mi, jnp.int32(_N))
            j = jnp.min(cand)
            jt = pl.multiple_of((j >> 7) << 7, 128)
            pltpu.sync_copy(x_hbm.at[row, :, pl.ds(jt, 128)], win)
            jm = jnp.full((16,), 0, jnp.int32) + (j - jt)
            wv = plsc.load_gather(win, [iota, jm])
            wout[0, pl.ds(0, _D)] = wv
            pltpu.sync_copy(wout, o_hbm.at[pl.ds(row, 1), :])
        cur = nxt


_mask_cid = functools.partial(
    pl.kernel,
    out_type=jax.ShapeDtypeStruct((_B, _D), jnp.float32),
    mesh=plsc.VectorSubcoreMesh(core_axis_name="c", subcore_axis_name="s"),
    compiler_params=pltpu.CompilerParams(
        needs_layout_passes=False, use_tc_tiling_on_sc=True),
    scratch_types=[
        pltpu.VMEM((_D, _CW), jnp.float32),
        pltpu.VMEM((_D, _CW), jnp.float32),
        pltpu.VMEM((_D, 128), jnp.float32),
        pltpu.VMEM((1, _D), jnp.float32),
        pltpu.SemaphoreType.DMA,
        pltpu.SemaphoreType.DMA,
    ],
)(_body)


@jax.jit
def kernel(inputs):
    xt = jnp.transpose(inputs, (0, 2, 1))
    return _mask_cid(xt)


# speedup vs baseline: 10.4725x; 1.2481x over previous
import functools

import jax
import jax.numpy as jnp
from jax import lax
from jax.experimental import pallas as pl
from jax.experimental.pallas import tpu as pltpu
from jax.experimental.pallas import tpu_sc as plsc

_B, _N, _D = 128, 8192, 16
_RSC = 64
_RTC = _B - _RSC
_NW = 32
_RPW = _RSC // _NW
_CW = 2048
_NCH = _N // _CW
_NG = _RPW * _NCH
_UNR = 2
_TCB = 8


def _sc_body(x_hbm, o_hbm, buf0, buf1, win, wout, sem0, sem1):
    wid = lax.axis_index("s") * 2 + lax.axis_index("c")
    iota = lax.iota(jnp.int32, 16)

    bufs = (buf0, buf1)
    sems = (sem0, sem1)

    def start(g):
        row = wid * _RPW + (g // _NCH)
        cap0 = (g % _NCH) * _CW
        return pltpu.async_copy(
            x_hbm.at[row, :, pl.ds(cap0, _CW)], bufs[g % 2], sems[g % 2])

    cur = start(0)
    bv = bi = None
    for g in range(_NG):
        nxt = start(g + 1) if g + 1 < _NG else None
        cur.wait()
        buf = bufs[g % 2]
        c = g % _NCH
        if c == 0:
            bv = [jnp.full((16,), -1.0, jnp.float32) for _ in range(_UNR)]
            bi = [jnp.zeros((16,), jnp.int32) for _ in range(_UNR)]
        cbase = c * _CW

        def grp(q, carry, buf=buf, cbase=cbase):
            sbv = list(carry[: _UNR])
            sbi = list(carry[_UNR:])
            for m in range(_UNR):
                vs = [buf[k, pl.ds(q * (16 * _UNR) + m * 16, 16)]
                      for k in range(_D)]
                sq = [v * v for v in vs]
                while len(sq) > 1:
                    sq = [sq[i] + sq[i + 1] for i in range(0, len(sq), 2)]
                acc = sq[0]
                idxv = iota + (cbase + m * 16) + q * (16 * _UNR)
                m_upd = acc > sbv[m]
                sbv[m] = jnp.where(m_upd, acc, sbv[m])
                sbi[m] = jnp.where(m_upd, idxv, sbi[m])
            return tuple(sbv) + tuple(sbi)

        res = lax.fori_loop(0, _CW // (16 * _UNR), grp,
                            tuple(bv) + tuple(bi))
        bv = list(res[: _UNR])
        bi = list(res[_UNR:])

        if c == _NCH - 1:
            row = wid * _RPW + (g // _NCH)
            mv, mi = bv[0], bi[0]
            for u in range(1, _UNR):
                take = (bv[u] > mv) | ((bv[u] == mv) & (bi[u] < mi))
                mv = jnp.where(take, bv[u], mv)
                mi = jnp.where(take, bi[u], mi)
            mx = jnp.max(mv)
            cand = jnp.where(mv == mx, mi, jnp.int32(_N))
            j = jnp.min(cand)
            jt = pl.multiple_of((j >> 7) << 7, 128)
            pltpu.sync_copy(x_hbm.at[row, :, pl.ds(jt, 128)], win)
            jm = jnp.full((16,), 0, jnp.int32) + (j - jt)
            wv = plsc.load_gather(win, [iota, jm])
            wout[0, pl.ds(0, _D)] = wv
            pltpu.sync_copy(wout, o_hbm.at[pl.ds(row, 1), :])
        cur = nxt


_sc_part = functools.partial(
    pl.kernel,
    out_type=jax.ShapeDtypeStruct((_RSC, _D), jnp.float32),
    mesh=plsc.VectorSubcoreMesh(core_axis_name="c", subcore_axis_name="s"),
    compiler_params=pltpu.CompilerParams(
        needs_layout_passes=False, use_tc_tiling_on_sc=True),
    scratch_types=[
        pltpu.VMEM((_D, _CW), jnp.float32),
        pltpu.VMEM((_D, _CW), jnp.float32),
        pltpu.VMEM((_D, 128), jnp.float32),
        pltpu.VMEM((1, _D), jnp.float32),
        pltpu.SemaphoreType.DMA,
        pltpu.SemaphoreType.DMA,
    ],
)(_sc_body)


def _tc_body(x_ref, o_ref):
    x = x_ref[...]
    n2 = jnp.sum(x * x, axis=1)
    m = jnp.max(n2, axis=1, keepdims=True)
    iota2 = lax.broadcasted_iota(jnp.int32, (_TCB, _N), 1)
    cand = jnp.where(n2 == m, iota2, jnp.int32(_N))
    j = jnp.min(cand, axis=1)
    mask3 = lax.broadcasted_iota(jnp.int32, (_TCB, _D, _N), 2) \
        == j[:, None, None]
    o_ref[...] = jnp.sum(jnp.where(mask3, x, 0.0), axis=2)


_tc_part = pl.pallas_call(
    _tc_body,
    out_shape=jax.ShapeDtypeStruct((_RTC, _D), jnp.float32),
    grid=(_RTC // _TCB,),
    in_specs=[pl.BlockSpec((_TCB, _D, _N),
                           lambda i: (_RSC // _TCB + i, 0, 0))],
    out_specs=pl.BlockSpec((_TCB, _D), lambda i: (i, 0)),
    compiler_params=pltpu.CompilerParams(
        dimension_semantics=("arbitrary",)),
)


@jax.jit
def kernel(inputs):
    xt = jnp.transpose(inputs, (0, 2, 1))
    sc_out = _sc_part(xt)
    tc_out = _tc_part(xt)
    return jnp.concatenate([sc_out, tc_out], axis=0)


# speedup vs baseline: 10.5062x; 1.0032x over previous
import functools

import jax
import jax.numpy as jnp
from jax import lax
from jax.experimental import pallas as pl
from jax.experimental.pallas import tpu as pltpu
from jax.experimental.pallas import tpu_sc as plsc

_B, _N, _D = 128, 8192, 16
_RSC = 64
_RTC = _B - _RSC
_NW = 32
_RPW = _RSC // _NW
_CW = 2048
_NCH = _N // _CW
_NG = _RPW * _NCH
_UNR = 2
_TCB = 8


def _sc_body(x_hbm, o_hbm, buf0, buf1, win, wout, sem0, sem1):
    wid = lax.axis_index("s") * 2 + lax.axis_index("c")
    iota = lax.iota(jnp.int32, 16)

    bufs = (buf0, buf1)
    sems = (sem0, sem1)

    def start(g):
        row = wid * _RPW + (g // _NCH)
        cap0 = (g % _NCH) * _CW
        return pltpu.async_copy(
            x_hbm.at[row, :, pl.ds(cap0, _CW)], bufs[g % 2], sems[g % 2])

    cur = start(0)
    bv = bi = None
    for g in range(_NG):
        nxt = start(g + 1) if g + 1 < _NG else None
        cur.wait()
        buf = bufs[g % 2]
        c = g % _NCH
        if c == 0:
            bv = [jnp.full((16,), -1.0, jnp.float32) for _ in range(_UNR)]
            bi = [jnp.zeros((16,), jnp.int32) for _ in range(_UNR)]
        cbase = c * _CW

        def grp(q, carry, buf=buf, cbase=cbase):
            sbv = list(carry[: _UNR])
            sbi = list(carry[_UNR:])
            for m in range(_UNR):
                vs = [buf[k, pl.ds(q * (16 * _UNR) + m * 16, 16)]
                      for k in range(_D)]
                sq = [v * v for v in vs]
                while len(sq) > 1:
                    sq = [sq[i] + sq[i + 1] for i in range(0, len(sq), 2)]
                acc = sq[0]
                idxv = iota + (cbase + m * 16) + q * (16 * _UNR)
                m_upd = acc > sbv[m]
                sbv[m] = jnp.where(m_upd, acc, sbv[m])
                sbi[m] = jnp.where(m_upd, idxv, sbi[m])
            return tuple(sbv) + tuple(sbi)

        res = lax.fori_loop(0, _CW // (16 * _UNR), grp,
                            tuple(bv) + tuple(bi))
        bv = list(res[: _UNR])
        bi = list(res[_UNR:])

        if c == _NCH - 1:
            row = wid * _RPW + (g // _NCH)
            mv, mi = bv[0], bi[0]
            for u in range(1, _UNR):
                take = (bv[u] > mv) | ((bv[u] == mv) & (bi[u] < mi))
                mv = jnp.where(take, bv[u], mv)
                mi = jnp.where(take, bi[u], mi)
            mx = jnp.max(mv)
            cand = jnp.where(mv == mx, mi, jnp.int32(_N))
            j = jnp.min(cand)
            jt = pl.multiple_of((j >> 7) << 7, 128)
            pltpu.sync_copy(x_hbm.at[row, :, pl.ds(jt, 128)], win)
            jm = jnp.full((16,), 0, jnp.int32) + (j - jt)
            wv = plsc.load_gather(win, [iota, jm])
            wout[0, pl.ds(0, _D)] = wv
            pltpu.sync_copy(wout, o_hbm.at[pl.ds(row, 1), :])
        cur = nxt


_sc_part = functools.partial(
    pl.kernel,
    out_type=jax.ShapeDtypeStruct((_RSC, _D), jnp.float32),
    mesh=plsc.VectorSubcoreMesh(core_axis_name="c", subcore_axis_name="s"),
    compiler_params=pltpu.CompilerParams(
        needs_layout_passes=False, use_tc_tiling_on_sc=True),
    scratch_types=[
        pltpu.VMEM((_D, _CW), jnp.float32),
        pltpu.VMEM((_D, _CW), jnp.float32),
        pltpu.VMEM((_D, 128), jnp.float32),
        pltpu.VMEM((1, _D), jnp.float32),
        pltpu.SemaphoreType.DMA,
        pltpu.SemaphoreType.DMA,
    ],
)(_sc_body)


def _tc_body(x_ref, o_ref):
    x = x_ref[...]
    n2 = jnp.sum(x * x, axis=1)
    m = jnp.max(n2, axis=1, keepdims=True)
    iota2 = lax.broadcasted_iota(jnp.int32, (_TCB, _N), 1)
    cand = jnp.where(n2 == m, iota2, jnp.int32(_N))
    j = jnp.min(cand, axis=1)
    oh = (iota2 == j[:, None]).astype(jnp.float32)
    o_ref[...] = jnp.einsum('bkn,bn->bk', x, oh,
                            preferred_element_type=jnp.float32)


_tc_part = pl.pallas_call(
    _tc_body,
    out_shape=jax.ShapeDtypeStruct((_RTC, _D), jnp.float32),
    grid=(_RTC // _TCB,),
    in_specs=[pl.BlockSpec((_TCB, _D, _N),
                           lambda i: (_RSC // _TCB + i, 0, 0))],
    out_specs=pl.BlockSpec((_TCB, _D), lambda i: (i, 0)),
    compiler_params=pltpu.CompilerParams(
        dimension_semantics=("arbitrary",)),
)


@jax.jit
def kernel(inputs):
    xt = jnp.transpose(inputs, (0, 2, 1))
    sc_out = _sc_part(xt)
    tc_out = _tc_part(xt)
    return jnp.concatenate([sc_out, tc_out], axis=0)
